# baseline (speedup 1.0000x reference)
import functools

import jax
import jax.numpy as jnp
import numpy as np
from jax import lax
from jax.experimental import pallas as pl
from jax.experimental.pallas import tpu as pltpu
from jax.experimental.pallas import tpu_sc as plsc

B = 16384
F = 26
D = 16
VOCAB_PER_FIELD = 100000
N = B * F
NC, NS = 2, 16
NW = NC * NS
PER_W = N // NW
CHUNK = 1664
CHUNKS = PER_W // CHUNK
D_IN = F * D
H1, H2 = 256, 128
BB = 1024


def _sc_gather(x_flat, emb_table, lin16, off_flat):
    mesh = plsc.VectorSubcoreMesh(core_axis_name="c", subcore_axis_name="s")

    @functools.partial(
        pl.kernel,
        out_type=(
            jax.ShapeDtypeStruct((N, D), jnp.float32),
            jax.ShapeDtypeStruct((N,), jnp.float32),
        ),
        mesh=mesh,
        scratch_types=(
            pltpu.VMEM((CHUNK,), jnp.int32),
            pltpu.VMEM((CHUNK,), jnp.int32),
            pltpu.VMEM((CHUNK,), jnp.int32),
            pltpu.VMEM((CHUNK, D), jnp.float32),
            pltpu.VMEM((CHUNK, 16), jnp.float32),
            pltpu.VMEM((CHUNK,), jnp.float32),
            pltpu.SemaphoreType.DMA,
            pltpu.SemaphoreType.DMA,
        ),
        compiler_params=pltpu.CompilerParams(use_tc_tiling_on_sc=False,
                                             needs_layout_passes=False),
    )
    def k(x_hbm, emb_hbm, lin_hbm, off_hbm, oute_hbm, outl_hbm,
          idxb, lidxb, offb, ebuf, lbuf, lvals, sem_e, sem_l):
        wid = lax.axis_index("s") * NC + lax.axis_index("c")
        base = pl.multiple_of(wid * PER_W, 8)
        pltpu.sync_copy(off_hbm, offb)
        lane_iota = lax.iota(jnp.int32, 16)

        def chunk(j, carry):
            s0 = pl.multiple_of(base + j * CHUNK, 8)
            pltpu.sync_copy(x_hbm.at[pl.ds(s0, CHUNK)], idxb)

            def add(i, c):
                s = pl.ds(pl.multiple_of(i * 16, 16), 16)
                xi = idxb[s] + offb[s]
                idxb[s] = xi
                lidxb[s] = lax.shift_right_logical(xi, 4)
                return c

            lax.fori_loop(0, CHUNK // 16, add, 0)
            ce = pltpu.async_copy(emb_hbm.at[idxb], ebuf, sem_e)
            cl = pltpu.async_copy(lin_hbm.at[lidxb], lbuf, sem_l)
            ce.wait()
            cl.wait()

            def sel(i, c):
                s = pl.ds(pl.multiple_of(i * 16, 16), 16)
                col = lax.bitwise_and(idxb[s], 15)
                row = lane_iota + i * 16
                lvals[s] = plsc.load_gather(lbuf, [row, col])
                return c

            lax.fori_loop(0, CHUNK // 16, sel, 0)
            pltpu.sync_copy(ebuf, oute_hbm.at[pl.ds(s0, CHUNK)])
            pltpu.sync_copy(lvals, outl_hbm.at[pl.ds(s0, CHUNK)])
            return carry

        lax.fori_loop(0, CHUNKS, chunk, 0)

    return k(x_flat, emb_table, lin16, off_flat)


def _tc_body(h_ref, lv_ref, w1_ref, b1_ref, w2_ref, b2_ref, w3_ref, b3_ref,
             lb_ref, s_ref, out_ref):
    h = h_ref[...]
    se = jnp.dot(h, s_ref[...], preferred_element_type=jnp.float32)
    inter = 0.5 * (jnp.sum(se * se, axis=1, keepdims=True)
                   - jnp.sum(h * h, axis=1, keepdims=True))
    ylin = jnp.sum(lv_ref[...], axis=1, keepdims=True) + lb_ref[...]
    a = jnp.dot(h, w1_ref[...], preferred_element_type=jnp.float32) + b1_ref[...]
    a = jnp.maximum(a, 0.0)
    a = jnp.dot(a, w2_ref[...], preferred_element_type=jnp.float32) + b2_ref[...]
    a = jnp.maximum(a, 0.0)
    yd = jnp.dot(a, w3_ref[...], preferred_element_type=jnp.float32) + b3_ref[...]
    out_ref[...] = yd + inter + ylin


def _tc_mlp(h, linv, W1, b1, W2, b2, W3, b3, lb, S):
    grid = (B // BB,)
    return pl.pallas_call(
        _tc_body,
        grid=grid,
        in_specs=[
            pl.BlockSpec((BB, D_IN), lambda i: (i, 0)),
            pl.BlockSpec((BB, F), lambda i: (i, 0)),
            pl.BlockSpec((D_IN, H1), lambda i: (0, 0)),
            pl.BlockSpec((1, H1), lambda i: (0, 0)),
            pl.BlockSpec((H1, H2), lambda i: (0, 0)),
            pl.BlockSpec((1, H2), lambda i: (0, 0)),
            pl.BlockSpec((H2, 1), lambda i: (0, 0)),
            pl.BlockSpec((1, 1), lambda i: (0, 0)),
            pl.BlockSpec((1, 1), lambda i: (0, 0)),
            pl.BlockSpec((D_IN, D), lambda i: (0, 0)),
        ],
        out_specs=pl.BlockSpec((BB, 1), lambda i: (i, 0)),
        out_shape=jax.ShapeDtypeStruct((B, 1), jnp.float32),
    )(h, linv, W1, b1, W2, b2, W3, b3, lb, S)


def kernel(x, emb_table, lin_table, lin_bias, W1, b1, W2, b2, W3, b3):
    x_flat = x.reshape(N)
    pos = np.arange(CHUNK, dtype=np.int64)
    off_flat = jnp.asarray(((pos % F) * VOCAB_PER_FIELD).astype(np.int32))
    lin16 = lin_table.reshape(-1, 16)
    oute, outl = _sc_gather(x_flat, emb_table, lin16, off_flat)
    h = oute.reshape(B, D_IN)
    linv = outl.reshape(B, F)
    s_mat = jnp.asarray(np.tile(np.eye(D, dtype=np.float32), (F, 1)))
    y = _tc_mlp(h, linv, W1, b1.reshape(1, H1), W2, b2.reshape(1, H2),
                W3, b3.reshape(1, 1), lin_bias.reshape(1, 1), s_mat)
    return y.reshape(B)
